# baseline (speedup 1.0000x reference)
import functools

import jax
import jax.numpy as jnp
from jax import lax
from jax.experimental import pallas as pl
from jax.experimental.pallas import tpu as pltpu
from jax.experimental.pallas import tpu_sc as plsc

BATCH = 16384
PF = 64
NC = 2
NS = 16
NW = NC * NS
B_PER_W = BATCH // NW
CHUNK = 128
N_CHUNKS = B_PER_W // CHUNK
N_GROUPS = B_PER_W // 16


def _sc_gmf_body(uid_hbm, iid_hbm, ut_hbm, it_hbm, w_hbm, b_hbm, out_hbm,
                 uidx_v, iidx_v, u_rows, v_rows, w_v, b_v, out_v, t_v, sem):
    wid = lax.axis_index("s") * NC + lax.axis_index("c")
    base = wid * B_PER_W

    for c in range(N_CHUNKS):
        pltpu.sync_copy(uid_hbm.at[pl.ds(base + c * CHUNK, CHUNK)], uidx_v.at[c])
        pltpu.sync_copy(iid_hbm.at[pl.ds(base + c * CHUNK, CHUNK)], iidx_v.at[c])
    pltpu.sync_copy(w_hbm, w_v)
    pltpu.sync_copy(b_hbm, b_v)

    copies = []
    for c in range(N_CHUNKS):
        copies.append(pltpu.async_copy(
            ut_hbm.at[uidx_v.at[c]], u_rows.at[pl.ds(c * CHUNK, CHUNK)], sem))
        copies.append(pltpu.async_copy(
            it_hbm.at[iidx_v.at[c]], v_rows.at[pl.ds(c * CHUNK, CHUNK)], sem))
    for cp in copies:
        cp.wait()

    bias = b_v[...][0]
    w_chunks = [w_v[pl.ds(c * 16, 16)] for c in range(PF // 16)]
    lanes = lax.iota(jnp.int32, 16)
    scat_idx = lanes * 16

    def group_body(g, carry):
        for b in range(16):
            row = g * 16 + b
            p = jnp.zeros((16,), jnp.float32)
            for c in range(PF // 16):
                uc = u_rows[row, pl.ds(c * 16, 16)]
                vc = v_rows[row, pl.ds(c * 16, 16)]
                p = p + (uc * vc) * w_chunks[c]
            plsc.store_scatter(t_v, [scat_idx + b], p)
        acc = t_v[pl.ds(0, 16)]
        for l in range(1, 16):
            acc = acc + t_v[pl.ds(l * 16, 16)]
        z = acc + bias
        out_v[pl.ds(g * 16, 16)] = 1.0 / (1.0 + jnp.exp(-z))
        return carry

    lax.fori_loop(0, N_GROUPS, group_body, 0, unroll=False)

    pltpu.sync_copy(out_v, out_hbm.at[pl.ds(base, B_PER_W)])


@functools.partial(jax.jit, static_argnames=())
def _gmf(uid, iid, user_table, item_table, w, b16):
    mesh = plsc.VectorSubcoreMesh(
        core_axis_name="c", subcore_axis_name="s", num_cores=NC, num_subcores=NS)
    fn = pl.kernel(
        _sc_gmf_body,
        out_type=jax.ShapeDtypeStruct((BATCH,), jnp.float32),
        mesh=mesh,
        scratch_types=[
            pltpu.VMEM((N_CHUNKS, CHUNK), jnp.int32),
            pltpu.VMEM((N_CHUNKS, CHUNK), jnp.int32),
            pltpu.VMEM((B_PER_W, PF), jnp.float32),
            pltpu.VMEM((B_PER_W, PF), jnp.float32),
            pltpu.VMEM((PF,), jnp.float32),
            pltpu.VMEM((16,), jnp.float32),
            pltpu.VMEM((B_PER_W,), jnp.float32),
            pltpu.VMEM((256,), jnp.float32),
            pltpu.SemaphoreType.DMA,
        ],
        compiler_params=pltpu.CompilerParams(
            needs_layout_passes=False, use_tc_tiling_on_sc=False),
    )
    return fn(uid, iid, user_table, item_table, w, b16)


def kernel(x, user_table, item_table, W_out, b_out):
    uid = x[:, 0].astype(jnp.int32)
    iid = x[:, 1].astype(jnp.int32)
    w = W_out.reshape(-1).astype(jnp.float32)
    b16 = jnp.broadcast_to(b_out.reshape(-1), (16,)).astype(jnp.float32)
    return _gmf(uid, iid, user_table, item_table, w, b16)
